# baseline (speedup 1.0000x reference)
import functools

import jax
import jax.numpy as jnp
from jax import lax
from jax.experimental import pallas as pl
from jax.experimental.pallas import tpu as pltpu
from jax.experimental.pallas import tpu_sc as plsc

_V = 1000000
_B = 16384
_D = 16
_NC = 2
_NS = 16
_NW = _NC * _NS
_BPW = _B // _NW
_IPW = 2 * _BPW
_CHUNK = 128
_NCHUNK = _IPW // _CHUNK

_TBLK = 2048


def _transpose_kernel(t_ref, out_ref):
    out_ref[...] = t_ref[...].T


def _dot_kernel(table_hbm, idx_hbm, out_hbm, idx_v, rows, out_v, sem):
    wid = lax.axis_index("s") * _NC + lax.axis_index("c")
    base = wid * _BPW

    pltpu.sync_copy(idx_hbm.at[wid], idx_v)

    copies = []
    for j in range(_NCHUNK):
        dst = rows.at[pl.ds(j * _CHUNK, _CHUNK)]
        copies.append(pltpu.async_copy(table_hbm.at[idx_v.at[j]], dst, sem))
    for c in copies:
        c.wait()

    lane = lax.iota(jnp.int32, 16)

    def tile_body(t, _):
        acc = jnp.zeros((16,), jnp.float32)
        for r in range(16):
            i = 2 * (t * 16 + r)
            p = rows[i] * rows[i + 1]
            s = jnp.sum(p)
            acc = jnp.where(lane == r, s, acc)
        out_v[pl.ds(t * 16, 16)] = acc
        return ()

    lax.fori_loop(0, _BPW // 16, tile_body, ())

    pltpu.sync_copy(out_v, out_hbm.at[pl.ds(base, _BPW)])


@jax.jit
def kernel(inputs, table):
    table_t = table.T

    dense = pl.pallas_call(
        _transpose_kernel,
        grid=((_V + _TBLK - 1) // _TBLK,),
        in_specs=[pl.BlockSpec((_D, _TBLK), lambda i: (0, i))],
        out_specs=pl.BlockSpec((_TBLK, _D), lambda i: (i, 0)),
        out_shape=jax.ShapeDtypeStruct((_V, _D), jnp.float32),
    )(table_t)

    idx3 = inputs.reshape(_NW, _NCHUNK, _CHUNK)

    k = functools.partial(
        pl.kernel,
        mesh=plsc.VectorSubcoreMesh(core_axis_name="c", subcore_axis_name="s"),
        compiler_params=pltpu.CompilerParams(
            needs_layout_passes=False, use_tc_tiling_on_sc=False),
        out_type=jax.ShapeDtypeStruct((_B,), jnp.float32),
        scratch_types=[
            pltpu.VMEM((_NCHUNK, _CHUNK), jnp.int32),
            pltpu.VMEM((_IPW, _D), jnp.float32),
            pltpu.VMEM((_BPW,), jnp.float32),
            pltpu.SemaphoreType.DMA,
        ],
    )(_dot_kernel)

    out = k(dense, idx3)
    return out.reshape(_B, 1)


# speedup vs baseline: 1.3397x; 1.3397x over previous
import functools

import jax
import jax.numpy as jnp
from jax import lax
from jax.experimental import pallas as pl
from jax.experimental.pallas import tpu as pltpu
from jax.experimental.pallas import tpu_sc as plsc

_V = 1000000
_B = 16384
_D = 16
_NC = 2
_NS = 16
_NW = _NC * _NS
_BPW = _B // _NW
_IPW = 2 * _BPW
_CHUNK = 128
_NCHUNK = _IPW // _CHUNK

_TBLK = 8192


def _transpose_kernel(t_ref, out_ref):
    x = t_ref[...]
    eye = jnp.eye(_D, dtype=jnp.float32)
    out_ref[...] = jax.lax.dot_general(
        x, eye, (((0,), (0,)), ((), ())),
        preferred_element_type=jnp.float32)


def _dot_kernel(table_hbm, idx_hbm, out_hbm, idx_v, rows, out_v, sem):
    wid = lax.axis_index("s") * _NC + lax.axis_index("c")
    base = wid * _BPW

    pltpu.sync_copy(idx_hbm.at[wid], idx_v)

    copies = []
    for j in range(_NCHUNK):
        dst = rows.at[pl.ds(j * _CHUNK, _CHUNK)]
        copies.append(pltpu.async_copy(table_hbm.at[idx_v.at[j]], dst, sem))
    for c in copies:
        c.wait()

    lane = lax.iota(jnp.int32, 16)

    def tile_body(t, _):
        acc = jnp.zeros((16,), jnp.float32)
        for r in range(16):
            i = 2 * (t * 16 + r)
            p = rows[i] * rows[i + 1]
            s = jnp.sum(p)
            acc = jnp.where(lane == r, s, acc)
        out_v[pl.ds(t * 16, 16)] = acc
        return ()

    lax.fori_loop(0, _BPW // 16, tile_body, ())

    pltpu.sync_copy(out_v, out_hbm.at[pl.ds(base, _BPW)])


@jax.jit
def kernel(inputs, table):
    table_t = table.T

    dense = pl.pallas_call(
        _transpose_kernel,
        grid=((_V + _TBLK - 1) // _TBLK,),
        in_specs=[pl.BlockSpec((_D, _TBLK), lambda i: (0, i))],
        out_specs=pl.BlockSpec((_TBLK, _D), lambda i: (i, 0)),
        out_shape=jax.ShapeDtypeStruct((_V, _D), jnp.float32),
    )(table_t)

    idx3 = inputs.reshape(_NW, _NCHUNK, _CHUNK)

    k = functools.partial(
        pl.kernel,
        mesh=plsc.VectorSubcoreMesh(core_axis_name="c", subcore_axis_name="s"),
        compiler_params=pltpu.CompilerParams(
            needs_layout_passes=False, use_tc_tiling_on_sc=False),
        out_type=jax.ShapeDtypeStruct((_B,), jnp.float32),
        scratch_types=[
            pltpu.VMEM((_NCHUNK, _CHUNK), jnp.int32),
            pltpu.VMEM((_IPW, _D), jnp.float32),
            pltpu.VMEM((_BPW,), jnp.float32),
            pltpu.SemaphoreType.DMA,
        ],
    )(_dot_kernel)

    out = k(dense, idx3)
    return out.reshape(_B, 1)


# speedup vs baseline: 1.6223x; 1.2110x over previous
import functools

import jax
import jax.numpy as jnp
from jax import lax
from jax.experimental import pallas as pl
from jax.experimental.pallas import tpu as pltpu
from jax.experimental.pallas import tpu_sc as plsc

_V = 1000000
_B = 16384
_D = 16
_NC = 2
_NS = 16
_NW = _NC * _NS
_BPW = _B // _NW
_IPW = 2 * _BPW
_CHUNK = 128
_NCHUNK = _IPW // _CHUNK

def _dot_kernel(table_hbm, idx_hbm, out_hbm, idx_v, rows, out_v, sem):
    wid = lax.axis_index("s") * _NC + lax.axis_index("c")
    base = wid * _BPW

    pltpu.sync_copy(idx_hbm.at[wid], idx_v)

    copies = []
    for j in range(_NCHUNK):
        dst = rows.at[pl.ds(j * _CHUNK, _CHUNK)]
        copies.append(pltpu.async_copy(table_hbm.at[idx_v.at[j]], dst, sem))
    for c in copies:
        c.wait()

    lane = lax.iota(jnp.int32, 16)

    def tile_body(t, _):
        acc = jnp.zeros((16,), jnp.float32)
        for r in range(16):
            i = 2 * (t * 16 + r)
            p = rows[i] * rows[i + 1]
            s = jnp.sum(p)
            acc = jnp.where(lane == r, s, acc)
        out_v[pl.ds(t * 16, 16)] = acc
        return ()

    lax.fori_loop(0, _BPW // 16, tile_body, ())

    pltpu.sync_copy(out_v, out_hbm.at[pl.ds(base, _BPW)])


@jax.jit
def kernel(inputs, table):
    idx3 = inputs.reshape(_NW, _NCHUNK, _CHUNK)

    k = functools.partial(
        pl.kernel,
        mesh=plsc.VectorSubcoreMesh(core_axis_name="c", subcore_axis_name="s"),
        compiler_params=pltpu.CompilerParams(
            needs_layout_passes=False, use_tc_tiling_on_sc=False),
        out_type=jax.ShapeDtypeStruct((_B,), jnp.float32),
        scratch_types=[
            pltpu.VMEM((_NCHUNK, _CHUNK), jnp.int32),
            pltpu.VMEM((_IPW, _D), jnp.float32),
            pltpu.VMEM((_BPW,), jnp.float32),
            pltpu.SemaphoreType.DMA,
        ],
    )(_dot_kernel)

    out = k(table, idx3)
    return out.reshape(_B, 1)


# speedup vs baseline: 1.6274x; 1.0031x over previous
import functools

import jax
import jax.numpy as jnp
from jax import lax
from jax.experimental import pallas as pl
from jax.experimental.pallas import tpu as pltpu
from jax.experimental.pallas import tpu_sc as plsc

_B = 16384
_D = 16
_NC = 2
_NS = 16
_NW = _NC * _NS
_BPW = _B // _NW
_IPW = 2 * _BPW
_CHUNK = 128
_NCHUNK = _IPW // _CHUNK


def _dot_kernel(table_hbm, idx_hbm, out_hbm, idx_v, rows, out_v, sem):
    wid = lax.axis_index("s") * _NC + lax.axis_index("c")
    base = wid * _BPW

    pltpu.sync_copy(idx_hbm.at[wid], idx_v)

    copies = []
    for j in range(_NCHUNK):
        dst = rows.at[pl.ds(j * _CHUNK, _CHUNK)]
        copies.append(pltpu.async_copy(table_hbm.at[idx_v.at[j]], dst, sem))
    for c in copies:
        c.wait()

    lane = lax.iota(jnp.int32, 16)

    def tile_body(t, _):
        acc = jnp.zeros((16,), jnp.float32)
        for r in range(16):
            i = 2 * (t * 16 + r)
            p = rows[i] * rows[i + 1]
            s = jnp.sum(p)
            acc = jnp.where(lane == r, s, acc)
        out_v[pl.ds(t * 16, 16)] = acc
        return ()

    lax.fori_loop(0, _BPW // 16, tile_body, ())

    pltpu.sync_copy(out_v, out_hbm.at[pl.ds(base, _BPW)])


@jax.jit
def kernel(inputs, table):
    idx3 = inputs.reshape(_NW, _NCHUNK, _CHUNK)

    k = functools.partial(
        pl.kernel,
        mesh=plsc.VectorSubcoreMesh(core_axis_name="c", subcore_axis_name="s"),
        compiler_params=pltpu.CompilerParams(
            needs_layout_passes=False, use_tc_tiling_on_sc=False),
        out_type=jax.ShapeDtypeStruct((_B,), jnp.float32),
        scratch_types=[
            pltpu.VMEM((_NCHUNK, _CHUNK), jnp.int32),
            pltpu.VMEM((_IPW, _D), jnp.float32),
            pltpu.VMEM((_BPW,), jnp.float32),
            pltpu.SemaphoreType.DMA,
        ],
    )(_dot_kernel)

    out = k(table, idx3)
    return out.reshape(_B, 1)
